# baseline (speedup 1.0000x reference)
import functools

import jax
import jax.numpy as jnp
from jax import lax
from jax.experimental import pallas as pl
from jax.experimental.pallas import tpu as pltpu
from jax.experimental.pallas import tpu_sc as plsc

TOKENS = 2048
HIDDEN = 1024
INTER = 2048
E = 8

B = 128
NP = TOKENS + E * B
NBL = NP // B
GBLK = 256
NW = 32


def _gate_body(x_ref, wg_ref, out_ref, xc_ref):
    st = lax.dot_general(
        wg_ref[...], x_ref[...],
        dimension_numbers=(((1,), (1,)), ((), ())),
        preferred_element_type=jnp.float32,
    )
    bv = st[0:1, :]
    bi = jnp.zeros((1, GBLK), jnp.int32)
    for e in range(1, E):
        c = st[e:e + 1, :] > bv
        bi = jnp.where(c, e, bi)
        bv = jnp.where(c, st[e:e + 1, :], bv)
    out_ref[0] = bi
    xc_ref[:, :HIDDEN] = x_ref[...]
    xc_ref[:, HIDDEN:] = x_ref[...]


def _gating(x, wg):
    n = TOKENS // GBLK
    out, xc = pl.pallas_call(
        _gate_body,
        grid=(n,),
        in_specs=[
            pl.BlockSpec((GBLK, HIDDEN), lambda w: (w, 0)),
            pl.BlockSpec((E, HIDDEN), lambda w: (0, 0)),
        ],
        out_specs=[
            pl.BlockSpec((1, 1, GBLK), lambda w: (w, 0, 0)),
            pl.BlockSpec((GBLK, 2 * HIDDEN), lambda w: (w, 0)),
        ],
        out_shape=[
            jax.ShapeDtypeStruct((n, 1, GBLK), jnp.int32),
            jax.ShapeDtypeStruct((TOKENS, 2 * HIDDEN), jnp.float32),
        ],
    )(x, wg)
    return out.reshape(TOKENS), xc


_DCH = 24
_DNCH = (NP // NW) // _DCH


@functools.cache
def _make_sc_dispatch():
    @functools.partial(
        pl.kernel,
        out_type=jax.ShapeDtypeStruct((NP, 2 * HIDDEN), jnp.float32),
        mesh=plsc.VectorSubcoreMesh(core_axis_name="c", subcore_axis_name="s"),
        scratch_types=[
            pltpu.VMEM((NP // NW,), jnp.int32),
            pltpu.VMEM((_DCH, 2 * HIDDEN), jnp.float32),
            pltpu.VMEM((_DCH, 2 * HIDDEN), jnp.float32),
            pltpu.SemaphoreType.DMA,
            pltpu.SemaphoreType.DMA,
            pltpu.SemaphoreType.DMA,
            pltpu.SemaphoreType.DMA,
        ],
    )
    def _sc_dispatch(x_hbm, gidx_hbm, out_hbm, idx_v, b0, b1, sg0, sg1,
                     sw0, sw1):
        bpw = NP // NW
        wid = lax.axis_index("s") * 2 + lax.axis_index("c")
        base = wid * bpw
        pltpu.sync_copy(gidx_hbm.at[pl.ds(base, bpw)], idx_v)
        bufs = (b0, b1)
        sg = (sg0, sg1)
        sw = (sw0, sw1)
        gops = [None, None]
        wops = [None, None]
        for c in range(_DNCH):
            b = c & 1
            if wops[b] is not None:
                wops[b].wait()
            gops[b] = pltpu.async_copy(
                x_hbm.at[idx_v.at[pl.ds(c * _DCH, _DCH)]], bufs[b], sg[b])
            if c >= 1:
                pb = (c - 1) & 1
                gops[pb].wait()
                wops[pb] = pltpu.async_copy(
                    bufs[pb], out_hbm.at[pl.ds(base + (c - 1) * _DCH, _DCH)],
                    sw[pb])
        lb = (_DNCH - 1) & 1
        gops[lb].wait()
        wops[lb] = pltpu.async_copy(
            bufs[lb], out_hbm.at[pl.ds(base + (_DNCH - 1) * _DCH, _DCH)],
            sw[lb])
        wops[(_DNCH - 2) & 1].wait()
        wops[lb].wait()

    return _sc_dispatch


_CCH = 16
_CNCH = (TOKENS // NW) // _CCH


@functools.cache
def _make_sc_combine():
    @functools.partial(
        pl.kernel,
        out_type=jax.ShapeDtypeStruct((TOKENS, INTER), jnp.float32),
        mesh=plsc.VectorSubcoreMesh(core_axis_name="c", subcore_axis_name="s"),
        scratch_types=[
            pltpu.VMEM((TOKENS // NW,), jnp.int32),
            pltpu.VMEM((_CCH, INTER), jnp.float32),
            pltpu.VMEM((_CCH, INTER), jnp.float32),
            pltpu.SemaphoreType.DMA,
            pltpu.SemaphoreType.DMA,
            pltpu.SemaphoreType.DMA,
            pltpu.SemaphoreType.DMA,
        ],
    )
    def _sc_combine(src_hbm, g2_hbm, out_hbm, idx_v, b0, b1, sg0, sg1,
                    sw0, sw1):
        bpw = TOKENS // NW
        wid = lax.axis_index("s") * 2 + lax.axis_index("c")
        base = wid * bpw
        pltpu.sync_copy(g2_hbm.at[pl.ds(base, bpw)], idx_v)
        bufs = (b0, b1)
        sg = (sg0, sg1)
        sw = (sw0, sw1)
        gops = [None, None]
        wops = [None, None]
        for c in range(_CNCH):
            b = c & 1
            if wops[b] is not None:
                wops[b].wait()
            gops[b] = pltpu.async_copy(
                src_hbm.at[idx_v.at[pl.ds(c * _CCH, _CCH)]], bufs[b], sg[b])
            if c >= 1:
                pb = (c - 1) & 1
                gops[pb].wait()
                wops[pb] = pltpu.async_copy(
                    bufs[pb], out_hbm.at[pl.ds(base + (c - 1) * _CCH, _CCH)],
                    sw[pb])
        lb = (_CNCH - 1) & 1
        gops[lb].wait()
        wops[lb] = pltpu.async_copy(
            bufs[lb], out_hbm.at[pl.ds(base + (_CNCH - 1) * _CCH, _CCH)],
            sw[lb])
        wops[(_CNCH - 2) & 1].wait()
        wops[lb].wait()

    return _sc_combine


def _dispatch_gather(x, gidx):
    return _make_sc_dispatch()(x, gidx)


def _combine_gather(src, g2):
    return _make_sc_combine()(src, g2)


def _moe_body(beid_ref, xs_ref, we_ref, out_ref):
    del beid_ref
    w = we_ref[0]
    out_ref[...] = lax.dot_general(
        xs_ref[...], w,
        dimension_numbers=(((1,), (1,)), ((), ())),
        preferred_element_type=jnp.float32,
    )


def _expert_matmul(xs, we, beid):
    grid_spec = pltpu.PrefetchScalarGridSpec(
        num_scalar_prefetch=1,
        grid=(NBL,),
        in_specs=[
            pl.BlockSpec((B, HIDDEN), lambda w, beid: (w, 0)),
            pl.BlockSpec((1, INTER, HIDDEN), lambda w, beid: (beid[w], 0, 0)),
        ],
        out_specs=pl.BlockSpec((B, INTER), lambda w, beid: (w, 0)),
    )
    return pl.pallas_call(
        _moe_body,
        grid_spec=grid_spec,
        out_shape=jax.ShapeDtypeStruct((NP, INTER), jnp.float32),
        compiler_params=pltpu.CompilerParams(
            dimension_semantics=("arbitrary",),
        ),
    )(beid, xs, we)


def kernel(x, Wg, We):
    eidx, xc = _gating(x, Wg)

    oh = (eidx[:, None] == jnp.arange(E, dtype=jnp.int32)[None, :]).astype(
        jnp.int32)
    ccum = jnp.cumsum(oh, axis=0)
    counts = ccum[-1]
    rank = jnp.take_along_axis(ccum, eidx[:, None], axis=1)[:, 0] - 1
    caps = ((counts + B - 1) // B) * B
    pstarts = jnp.concatenate(
        [jnp.zeros((1,), jnp.int32), jnp.cumsum(caps)])[:E]
    ppos = pstarts[eidx] + rank
    gidx = jnp.zeros((NP,), jnp.int32).at[ppos].set(
        jnp.arange(TOKENS, dtype=jnp.int32))
    beid = (jnp.searchsorted(
        pstarts, jnp.arange(NBL, dtype=jnp.int32) * B, side="right")
        - 1).astype(jnp.int32)

    xs = _dispatch_gather(xc, gidx)
    out_sorted = _expert_matmul(xs, We, beid)
    return _combine_gather(out_sorted, ppos)


# speedup vs baseline: 1.5861x; 1.5861x over previous
import functools

import jax
import jax.numpy as jnp
from jax import lax
from jax.experimental import pallas as pl
from jax.experimental.pallas import tpu as pltpu
from jax.experimental.pallas import tpu_sc as plsc

TOKENS = 2048
HIDDEN = 1024
INTER = 2048
E = 8

B = 128
NP = TOKENS + E * B
NBL = NP // B
GBLK = 256
NW = 32


def _gate_body(x_ref, wg_ref, out_ref):
    st = lax.dot_general(
        wg_ref[...], x_ref[...],
        dimension_numbers=(((1,), (1,)), ((), ())),
        preferred_element_type=jnp.float32,
    )
    bv = st[0:1, :]
    bi = jnp.zeros((1, GBLK), jnp.int32)
    for e in range(1, E):
        c = st[e:e + 1, :] > bv
        bi = jnp.where(c, e, bi)
        bv = jnp.where(c, st[e:e + 1, :], bv)
    out_ref[0] = bi


def _gating(x, wg):
    n = TOKENS // GBLK
    out = pl.pallas_call(
        _gate_body,
        grid=(n,),
        in_specs=[
            pl.BlockSpec((GBLK, HIDDEN), lambda w: (w, 0)),
            pl.BlockSpec((E, HIDDEN), lambda w: (0, 0)),
        ],
        out_specs=pl.BlockSpec((1, 1, GBLK), lambda w: (w, 0, 0)),
        out_shape=jax.ShapeDtypeStruct((n, 1, GBLK), jnp.int32),
    )(x, wg)
    return out.reshape(TOKENS)


_DCH = 24
_DNCH = (NP // NW) // _DCH


@functools.cache
def _make_sc_dispatch():
    @functools.partial(
        pl.kernel,
        out_type=jax.ShapeDtypeStruct((NP, HIDDEN), jnp.float32),
        mesh=plsc.VectorSubcoreMesh(core_axis_name="c", subcore_axis_name="s"),
        scratch_types=[
            pltpu.VMEM((NP // NW,), jnp.int32),
            pltpu.VMEM((_DCH, HIDDEN), jnp.float32),
            pltpu.VMEM((_DCH, HIDDEN), jnp.float32),
            pltpu.SemaphoreType.DMA,
            pltpu.SemaphoreType.DMA,
            pltpu.SemaphoreType.DMA,
            pltpu.SemaphoreType.DMA,
        ],
    )
    def _sc_dispatch(x_hbm, gidx_hbm, out_hbm, idx_v, b0, b1, sg0, sg1,
                     sw0, sw1):
        bpw = NP // NW
        wid = lax.axis_index("s") * 2 + lax.axis_index("c")
        base = wid * bpw
        pltpu.sync_copy(gidx_hbm.at[pl.ds(base, bpw)], idx_v)
        bufs = (b0, b1)
        sg = (sg0, sg1)
        sw = (sw0, sw1)
        gops = [None, None]
        wops = [None, None]
        for c in range(_DNCH):
            b = c & 1
            if wops[b] is not None:
                wops[b].wait()
            gops[b] = pltpu.async_copy(
                x_hbm.at[idx_v.at[pl.ds(c * _DCH, _DCH)]], bufs[b], sg[b])
            if c >= 1:
                pb = (c - 1) & 1
                gops[pb].wait()
                wops[pb] = pltpu.async_copy(
                    bufs[pb], out_hbm.at[pl.ds(base + (c - 1) * _DCH, _DCH)],
                    sw[pb])
        lb = (_DNCH - 1) & 1
        gops[lb].wait()
        wops[lb] = pltpu.async_copy(
            bufs[lb], out_hbm.at[pl.ds(base + (_DNCH - 1) * _DCH, _DCH)],
            sw[lb])
        wops[(_DNCH - 2) & 1].wait()
        wops[lb].wait()

    return _sc_dispatch


_CCH = 16
_CNCH = (TOKENS // NW) // _CCH


@functools.cache
def _make_sc_combine():
    @functools.partial(
        pl.kernel,
        out_type=jax.ShapeDtypeStruct((TOKENS, INTER), jnp.float32),
        mesh=plsc.VectorSubcoreMesh(core_axis_name="c", subcore_axis_name="s"),
        scratch_types=[
            pltpu.VMEM((TOKENS // NW,), jnp.int32),
            pltpu.VMEM((_CCH, INTER), jnp.float32),
            pltpu.VMEM((_CCH, INTER), jnp.float32),
            pltpu.SemaphoreType.DMA,
            pltpu.SemaphoreType.DMA,
            pltpu.SemaphoreType.DMA,
            pltpu.SemaphoreType.DMA,
        ],
    )
    def _sc_combine(src_hbm, g2_hbm, out_hbm, idx_v, b0, b1, sg0, sg1,
                    sw0, sw1):
        bpw = TOKENS // NW
        wid = lax.axis_index("s") * 2 + lax.axis_index("c")
        base = wid * bpw
        pltpu.sync_copy(g2_hbm.at[pl.ds(base, bpw)], idx_v)
        bufs = (b0, b1)
        sg = (sg0, sg1)
        sw = (sw0, sw1)
        gops = [None, None]
        wops = [None, None]
        for c in range(_CNCH):
            b = c & 1
            if wops[b] is not None:
                wops[b].wait()
            gops[b] = pltpu.async_copy(
                src_hbm.at[idx_v.at[pl.ds(c * _CCH, _CCH)]], bufs[b], sg[b])
            if c >= 1:
                pb = (c - 1) & 1
                gops[pb].wait()
                wops[pb] = pltpu.async_copy(
                    bufs[pb], out_hbm.at[pl.ds(base + (c - 1) * _CCH, _CCH)],
                    sw[pb])
        lb = (_CNCH - 1) & 1
        gops[lb].wait()
        wops[lb] = pltpu.async_copy(
            bufs[lb], out_hbm.at[pl.ds(base + (_CNCH - 1) * _CCH, _CCH)],
            sw[lb])
        wops[(_CNCH - 2) & 1].wait()
        wops[lb].wait()

    return _sc_combine


def _dispatch_gather(x, gidx):
    return _make_sc_dispatch()(x, gidx)


def _combine_gather(src, g2):
    return _make_sc_combine()(src, g2)


def _moe_body(beid_ref, xs_ref, we_ref, out_ref):
    del beid_ref
    w = we_ref[0]
    out_ref[...] = lax.dot_general(
        xs_ref[...], w,
        dimension_numbers=(((1,), (1,)), ((), ())),
        preferred_element_type=jnp.float32,
    )


def _expert_matmul(xs, we, beid):
    grid_spec = pltpu.PrefetchScalarGridSpec(
        num_scalar_prefetch=1,
        grid=(NBL,),
        in_specs=[
            pl.BlockSpec((B, HIDDEN), lambda w, beid: (w, 0)),
            pl.BlockSpec((1, INTER, HIDDEN), lambda w, beid: (beid[w], 0, 0)),
        ],
        out_specs=pl.BlockSpec((B, INTER), lambda w, beid: (w, 0)),
    )
    return pl.pallas_call(
        _moe_body,
        grid_spec=grid_spec,
        out_shape=jax.ShapeDtypeStruct((NP, INTER), jnp.float32),
        compiler_params=pltpu.CompilerParams(
            dimension_semantics=("arbitrary",),
        ),
    )(beid, xs, we)


def kernel(x, Wg, We):
    eidx = _gating(x, Wg)

    oh = (eidx[:, None] == jnp.arange(E, dtype=jnp.int32)[None, :]).astype(
        jnp.int32)
    ccum = jnp.cumsum(oh, axis=0)
    counts = ccum[-1]
    rank = jnp.take_along_axis(ccum, eidx[:, None], axis=1)[:, 0] - 1
    caps = ((counts + B - 1) // B) * B
    pstarts = jnp.concatenate(
        [jnp.zeros((1,), jnp.int32), jnp.cumsum(caps)])[:E]
    ppos = pstarts[eidx] + rank
    gidx = (jnp.arange(NP, dtype=jnp.int32) % TOKENS).at[ppos].set(
        jnp.arange(TOKENS, dtype=jnp.int32))
    beid = (jnp.searchsorted(
        pstarts, jnp.arange(NBL, dtype=jnp.int32) * B, side="right")
        - 1).astype(jnp.int32)

    xs = _dispatch_gather(x, gidx)
    out_sorted = _expert_matmul(xs, We, beid)
    return _combine_gather(out_sorted, ppos)


# speedup vs baseline: 1.7133x; 1.0802x over previous
import functools

import jax
import jax.numpy as jnp
from jax import lax
from jax.experimental import pallas as pl
from jax.experimental.pallas import tpu as pltpu
from jax.experimental.pallas import tpu_sc as plsc

TOKENS = 2048
HIDDEN = 1024
INTER = 2048
E = 8

B = 128
NP = TOKENS + E * B
NBL = NP // B
GBLK = 256
NW = 32


def _gate_body(x_ref, wg_ref, out_ref):
    st = lax.dot_general(
        wg_ref[...], x_ref[...],
        dimension_numbers=(((1,), (1,)), ((), ())),
        preferred_element_type=jnp.float32,
    )
    bv = st[0:1, :]
    bi = jnp.zeros((1, GBLK), jnp.int32)
    for e in range(1, E):
        c = st[e:e + 1, :] > bv
        bi = jnp.where(c, e, bi)
        bv = jnp.where(c, st[e:e + 1, :], bv)
    out_ref[0] = bi


def _gating(x, wg):
    n = TOKENS // GBLK
    out = pl.pallas_call(
        _gate_body,
        grid=(n,),
        in_specs=[
            pl.BlockSpec((GBLK, HIDDEN), lambda w: (w, 0)),
            pl.BlockSpec((E, HIDDEN), lambda w: (0, 0)),
        ],
        out_specs=pl.BlockSpec((1, 1, GBLK), lambda w: (w, 0, 0)),
        out_shape=jax.ShapeDtypeStruct((n, 1, GBLK), jnp.int32),
    )(x, wg)
    return out.reshape(TOKENS)


_DCH = 24
_DNCH = (NP // NW) // _DCH


@functools.cache
def _make_sc_dispatch():
    @functools.partial(
        pl.kernel,
        out_type=jax.ShapeDtypeStruct((NP, HIDDEN), jnp.float32),
        mesh=plsc.VectorSubcoreMesh(core_axis_name="c", subcore_axis_name="s"),
        scratch_types=[
            pltpu.VMEM((NP // NW,), jnp.int32),
            pltpu.VMEM((_DCH, HIDDEN), jnp.float32),
            pltpu.VMEM((_DCH, HIDDEN), jnp.float32),
            pltpu.SemaphoreType.DMA,
            pltpu.SemaphoreType.DMA,
            pltpu.SemaphoreType.DMA,
            pltpu.SemaphoreType.DMA,
        ],
    )
    def _sc_dispatch(x_hbm, gidx_hbm, out_hbm, idx_v, b0, b1, sg0, sg1,
                     sw0, sw1):
        bpw = NP // NW
        wid = lax.axis_index("s") * 2 + lax.axis_index("c")
        base = wid * bpw
        pltpu.sync_copy(gidx_hbm.at[pl.ds(base, bpw)], idx_v)
        bufs = (b0, b1)
        sg = (sg0, sg1)
        sw = (sw0, sw1)
        gops = [None, None]
        wops = [None, None]
        for c in range(_DNCH):
            b = c & 1
            if wops[b] is not None:
                wops[b].wait()
            gops[b] = pltpu.async_copy(
                x_hbm.at[idx_v.at[pl.ds(c * _DCH, _DCH)]], bufs[b], sg[b])
            if c >= 1:
                pb = (c - 1) & 1
                gops[pb].wait()
                wops[pb] = pltpu.async_copy(
                    bufs[pb], out_hbm.at[pl.ds(base + (c - 1) * _DCH, _DCH)],
                    sw[pb])
        lb = (_DNCH - 1) & 1
        gops[lb].wait()
        wops[lb] = pltpu.async_copy(
            bufs[lb], out_hbm.at[pl.ds(base + (_DNCH - 1) * _DCH, _DCH)],
            sw[lb])
        wops[(_DNCH - 2) & 1].wait()
        wops[lb].wait()

    return _sc_dispatch


_CCH = 16
_CNCH = (TOKENS // NW) // _CCH


@functools.cache
def _make_sc_combine():
    @functools.partial(
        pl.kernel,
        out_type=jax.ShapeDtypeStruct((TOKENS, INTER), jnp.float32),
        mesh=plsc.VectorSubcoreMesh(core_axis_name="c", subcore_axis_name="s"),
        scratch_types=[
            pltpu.VMEM((TOKENS // NW,), jnp.int32),
            pltpu.VMEM((_CCH, INTER), jnp.float32),
            pltpu.VMEM((_CCH, INTER), jnp.float32),
            pltpu.SemaphoreType.DMA,
            pltpu.SemaphoreType.DMA,
            pltpu.SemaphoreType.DMA,
            pltpu.SemaphoreType.DMA,
        ],
    )
    def _sc_combine(src_hbm, g2_hbm, out_hbm, idx_v, b0, b1, sg0, sg1,
                    sw0, sw1):
        bpw = TOKENS // NW
        wid = lax.axis_index("s") * 2 + lax.axis_index("c")
        base = wid * bpw
        pltpu.sync_copy(g2_hbm.at[pl.ds(base, bpw)], idx_v)
        bufs = (b0, b1)
        sg = (sg0, sg1)
        sw = (sw0, sw1)
        gops = [None, None]
        wops = [None, None]
        for c in range(_CNCH):
            b = c & 1
            if wops[b] is not None:
                wops[b].wait()
            gops[b] = pltpu.async_copy(
                src_hbm.at[idx_v.at[pl.ds(c * _CCH, _CCH)]], bufs[b], sg[b])
            if c >= 1:
                pb = (c - 1) & 1
                gops[pb].wait()
                wops[pb] = pltpu.async_copy(
                    bufs[pb], out_hbm.at[pl.ds(base + (c - 1) * _CCH, _CCH)],
                    sw[pb])
        lb = (_CNCH - 1) & 1
        gops[lb].wait()
        wops[lb] = pltpu.async_copy(
            bufs[lb], out_hbm.at[pl.ds(base + (_CNCH - 1) * _CCH, _CCH)],
            sw[lb])
        wops[(_CNCH - 2) & 1].wait()
        wops[lb].wait()

    return _sc_combine


def _dispatch_gather(x, gidx):
    return _make_sc_dispatch()(x, gidx)


def _combine_gather(src, g2):
    return _make_sc_combine()(src, g2)


def _moe_body(beid_ref, rstart_ref, xs_ref, we_ref, out_ref, wbuf, s0, s1,
              s2):
    w = pl.program_id(0)
    sems = (s0, s1, s2)

    def dma(r):
        return pltpu.make_async_copy(we_ref.at[r], wbuf.at[r % 3],
                                     sems[r % 3])

    @pl.when(w == 0)
    def _():
        dma(0).start()
        dma(1).start()

    for r in range(E):
        @pl.when(w == rstart_ref[r])
        def _(r=r):
            dma(r).wait()
            if r + 2 < E:
                dma(r + 2).start()

    pb = beid_ref[w] % 3
    out_ref[...] = lax.dot_general(
        xs_ref[...], wbuf[pb],
        dimension_numbers=(((1,), (1,)), ((), ())),
        preferred_element_type=jnp.float32,
    )


def _expert_matmul(xs, we, beid, rstart):
    grid_spec = pltpu.PrefetchScalarGridSpec(
        num_scalar_prefetch=2,
        grid=(NBL,),
        in_specs=[
            pl.BlockSpec((B, HIDDEN), lambda w, beid, rstart: (w, 0)),
            pl.BlockSpec(memory_space=pl.ANY),
        ],
        out_specs=pl.BlockSpec((B, INTER), lambda w, beid, rstart: (w, 0)),
        scratch_shapes=[
            pltpu.VMEM((3, INTER, HIDDEN), jnp.float32),
            pltpu.SemaphoreType.DMA,
            pltpu.SemaphoreType.DMA,
            pltpu.SemaphoreType.DMA,
        ],
    )
    return pl.pallas_call(
        _moe_body,
        grid_spec=grid_spec,
        out_shape=jax.ShapeDtypeStruct((NP, INTER), jnp.float32),
        compiler_params=pltpu.CompilerParams(
            dimension_semantics=("arbitrary",),
        ),
    )(beid, rstart, xs, we)


def kernel(x, Wg, We):
    eidx = _gating(x, Wg)

    oh = (eidx[:, None] == jnp.arange(E, dtype=jnp.int32)[None, :]).astype(
        jnp.int32)
    ccum = jnp.cumsum(oh, axis=0)
    counts = ccum[-1]
    rank = jnp.take_along_axis(ccum, eidx[:, None], axis=1)[:, 0] - 1
    caps = ((counts + B - 1) // B) * B
    pstarts = jnp.concatenate(
        [jnp.zeros((1,), jnp.int32), jnp.cumsum(caps)])[:E]
    ppos = pstarts[eidx] + rank
    gidx = (jnp.arange(NP, dtype=jnp.int32) % TOKENS).at[ppos].set(
        jnp.arange(TOKENS, dtype=jnp.int32))
    beid = (jnp.searchsorted(
        pstarts, jnp.arange(NBL, dtype=jnp.int32) * B, side="right")
        - 1).astype(jnp.int32)

    rstart = (pstarts // B).astype(jnp.int32)
    xs = _dispatch_gather(x, gidx)
    out_sorted = _expert_matmul(xs, We, beid, rstart)
    return _combine_gather(out_sorted, ppos)


# speedup vs baseline: 1.7864x; 1.0427x over previous
import functools

import jax
import jax.numpy as jnp
from jax import lax
from jax.experimental import pallas as pl
from jax.experimental.pallas import tpu as pltpu
from jax.experimental.pallas import tpu_sc as plsc

TOKENS = 2048
HIDDEN = 1024
INTER = 2048
E = 8

B = 128
NP = TOKENS + E * B
NBL = NP // B
GBLK = 256
NW = 32


def _gate_body(x_ref, wg_ref, out_ref):
    st = lax.dot_general(
        wg_ref[...], x_ref[...],
        dimension_numbers=(((1,), (1,)), ((), ())),
        preferred_element_type=jnp.float32,
    )
    bv = st[0:1, :]
    bi = jnp.zeros((1, GBLK), jnp.int32)
    for e in range(1, E):
        c = st[e:e + 1, :] > bv
        bi = jnp.where(c, e, bi)
        bv = jnp.where(c, st[e:e + 1, :], bv)
    out_ref[0] = bi


def _gating(x, wg):
    n = TOKENS // GBLK
    out = pl.pallas_call(
        _gate_body,
        grid=(n,),
        in_specs=[
            pl.BlockSpec((GBLK, HIDDEN), lambda w: (w, 0)),
            pl.BlockSpec((E, HIDDEN), lambda w: (0, 0)),
        ],
        out_specs=pl.BlockSpec((1, 1, GBLK), lambda w: (w, 0, 0)),
        out_shape=jax.ShapeDtypeStruct((n, 1, GBLK), jnp.int32),
    )(x, wg)
    return out.reshape(TOKENS)


_DCH = 24
_DNCH = (NP // NW) // _DCH


@functools.cache
def _make_sc_dispatch():
    @functools.partial(
        pl.kernel,
        out_type=jax.ShapeDtypeStruct((NP, HIDDEN), jnp.float32),
        mesh=plsc.VectorSubcoreMesh(core_axis_name="c", subcore_axis_name="s"),
        scratch_types=[
            pltpu.VMEM((NP // NW,), jnp.int32),
            pltpu.VMEM((_DCH, HIDDEN), jnp.float32),
            pltpu.VMEM((_DCH, HIDDEN), jnp.float32),
            pltpu.SemaphoreType.DMA,
            pltpu.SemaphoreType.DMA,
            pltpu.SemaphoreType.DMA,
            pltpu.SemaphoreType.DMA,
        ],
    )
    def _sc_dispatch(x_hbm, gidx_hbm, out_hbm, idx_v, b0, b1, sg0, sg1,
                     sw0, sw1):
        bpw = NP // NW
        wid = lax.axis_index("s") * 2 + lax.axis_index("c")
        base = wid * bpw
        pltpu.sync_copy(gidx_hbm.at[pl.ds(base, bpw)], idx_v)
        bufs = (b0, b1)
        sg = (sg0, sg1)
        sw = (sw0, sw1)
        gops = [None, None]
        wops = [None, None]
        for c in range(_DNCH):
            b = c & 1
            if wops[b] is not None:
                wops[b].wait()
            gops[b] = pltpu.async_copy(
                x_hbm.at[idx_v.at[pl.ds(c * _DCH, _DCH)]], bufs[b], sg[b])
            if c >= 1:
                pb = (c - 1) & 1
                gops[pb].wait()
                wops[pb] = pltpu.async_copy(
                    bufs[pb], out_hbm.at[pl.ds(base + (c - 1) * _DCH, _DCH)],
                    sw[pb])
        lb = (_DNCH - 1) & 1
        gops[lb].wait()
        wops[lb] = pltpu.async_copy(
            bufs[lb], out_hbm.at[pl.ds(base + (_DNCH - 1) * _DCH, _DCH)],
            sw[lb])
        wops[(_DNCH - 2) & 1].wait()
        wops[lb].wait()

    return _sc_dispatch


_CCH = 16
_CNCH = (TOKENS // NW) // _CCH


@functools.cache
def _make_sc_combine():
    @functools.partial(
        pl.kernel,
        out_type=jax.ShapeDtypeStruct((TOKENS, INTER), jnp.float32),
        mesh=plsc.VectorSubcoreMesh(core_axis_name="c", subcore_axis_name="s"),
        scratch_types=[
            pltpu.VMEM((TOKENS // NW,), jnp.int32),
            pltpu.VMEM((_CCH, INTER), jnp.float32),
            pltpu.VMEM((_CCH, INTER), jnp.float32),
            pltpu.SemaphoreType.DMA,
            pltpu.SemaphoreType.DMA,
            pltpu.SemaphoreType.DMA,
            pltpu.SemaphoreType.DMA,
        ],
    )
    def _sc_combine(src_hbm, g2_hbm, out_hbm, idx_v, b0, b1, sg0, sg1,
                    sw0, sw1):
        bpw = TOKENS // NW
        wid = lax.axis_index("s") * 2 + lax.axis_index("c")
        base = wid * bpw
        pltpu.sync_copy(g2_hbm.at[pl.ds(base, bpw)], idx_v)
        bufs = (b0, b1)
        sg = (sg0, sg1)
        sw = (sw0, sw1)
        gops = [None, None]
        wops = [None, None]
        for c in range(_CNCH):
            b = c & 1
            if wops[b] is not None:
                wops[b].wait()
            gops[b] = pltpu.async_copy(
                src_hbm.at[idx_v.at[pl.ds(c * _CCH, _CCH)]], bufs[b], sg[b])
            if c >= 1:
                pb = (c - 1) & 1
                gops[pb].wait()
                wops[pb] = pltpu.async_copy(
                    bufs[pb], out_hbm.at[pl.ds(base + (c - 1) * _CCH, _CCH)],
                    sw[pb])
        lb = (_CNCH - 1) & 1
        gops[lb].wait()
        wops[lb] = pltpu.async_copy(
            bufs[lb], out_hbm.at[pl.ds(base + (_CNCH - 1) * _CCH, _CCH)],
            sw[lb])
        wops[(_CNCH - 2) & 1].wait()
        wops[lb].wait()

    return _sc_combine


def _dispatch_gather(x, gidx):
    return _make_sc_dispatch()(x, gidx)


def _combine_gather(src, g2):
    return _make_sc_combine()(src, g2)


def _moe_body(beid_ref, rstart_ref, xs_ref, we_ref, out_ref, wbuf, s0, s1,
              s2, s3):
    w = pl.program_id(0)
    sems = (s0, s1, s2, s3)

    def dma(r):
        return pltpu.make_async_copy(we_ref.at[r], wbuf.at[r % 4],
                                     sems[r % 4])

    @pl.when(w == 0)
    def _():
        dma(0).start()
        dma(1).start()
        dma(2).start()

    for r in range(E):
        @pl.when(w == rstart_ref[r])
        def _(r=r):
            dma(r).wait()
            if r + 3 < E:
                dma(r + 3).start()

    pb = beid_ref[w] % 4
    out_ref[...] = lax.dot_general(
        xs_ref[...], wbuf[pb],
        dimension_numbers=(((1,), (1,)), ((), ())),
        preferred_element_type=jnp.float32,
    )


def _expert_matmul(xs, we, beid, rstart):
    grid_spec = pltpu.PrefetchScalarGridSpec(
        num_scalar_prefetch=2,
        grid=(NBL,),
        in_specs=[
            pl.BlockSpec((B, HIDDEN), lambda w, beid, rstart: (w, 0)),
            pl.BlockSpec(memory_space=pl.ANY),
        ],
        out_specs=pl.BlockSpec((B, INTER), lambda w, beid, rstart: (w, 0)),
        scratch_shapes=[
            pltpu.VMEM((4, INTER, HIDDEN), jnp.float32),
            pltpu.SemaphoreType.DMA,
            pltpu.SemaphoreType.DMA,
            pltpu.SemaphoreType.DMA,
            pltpu.SemaphoreType.DMA,
        ],
    )
    return pl.pallas_call(
        _moe_body,
        grid_spec=grid_spec,
        out_shape=jax.ShapeDtypeStruct((NP, INTER), jnp.float32),
        compiler_params=pltpu.CompilerParams(
            dimension_semantics=("arbitrary",),
        ),
    )(beid, rstart, xs, we)


def kernel(x, Wg, We):
    eidx = _gating(x, Wg)

    oh = (eidx[:, None] == jnp.arange(E, dtype=jnp.int32)[None, :]).astype(
        jnp.int32)
    ccum = jnp.cumsum(oh, axis=0)
    counts = ccum[-1]
    rank = jnp.sum(oh * ccum, axis=1) - 1
    caps = ((counts + B - 1) // B) * B
    pstarts = jnp.concatenate(
        [jnp.zeros((1,), jnp.int32), jnp.cumsum(caps)])[:E]
    ppos = jnp.sum(oh * pstarts[None, :], axis=1) + rank
    gidx = (jnp.arange(NP, dtype=jnp.int32) % TOKENS).at[ppos].set(
        jnp.arange(TOKENS, dtype=jnp.int32))
    beid = (jnp.sum(
        (jnp.arange(NBL, dtype=jnp.int32)[:, None] * B) >= pstarts[None, :],
        axis=1) - 1).astype(jnp.int32)

    rstart = (pstarts // B).astype(jnp.int32)
    xs = _dispatch_gather(x, gidx)
    out_sorted = _expert_matmul(xs, We, beid, rstart)
    return _combine_gather(out_sorted, ppos)


# speedup vs baseline: 1.9910x; 1.1145x over previous
import functools

import jax
import jax.numpy as jnp
from jax import lax
from jax.experimental import pallas as pl
from jax.experimental.pallas import tpu as pltpu
from jax.experimental.pallas import tpu_sc as plsc

TOKENS = 2048
HIDDEN = 1024
INTER = 2048
E = 8

B = 128
NP = TOKENS + E * B
NBL = NP // B
GBLK = 256
NW = 32


def _gate_body(x_ref, wg_ref, out_ref):
    st = lax.dot_general(
        wg_ref[...], x_ref[...],
        dimension_numbers=(((1,), (1,)), ((), ())),
        preferred_element_type=jnp.float32,
    )
    bv = st[0:1, :]
    bi = jnp.zeros((1, GBLK), jnp.int32)
    for e in range(1, E):
        c = st[e:e + 1, :] > bv
        bi = jnp.where(c, e, bi)
        bv = jnp.where(c, st[e:e + 1, :], bv)
    out_ref[0] = bi


def _gating(x, wg):
    n = TOKENS // GBLK
    out = pl.pallas_call(
        _gate_body,
        grid=(n,),
        in_specs=[
            pl.BlockSpec((GBLK, HIDDEN), lambda w: (w, 0)),
            pl.BlockSpec((E, HIDDEN), lambda w: (0, 0)),
        ],
        out_specs=pl.BlockSpec((1, 1, GBLK), lambda w: (w, 0, 0)),
        out_shape=jax.ShapeDtypeStruct((n, 1, GBLK), jnp.int32),
    )(x, wg)
    return out.reshape(TOKENS)


_DCH = 32


@functools.cache
def _make_sc_dispatch():
    @functools.partial(
        pl.kernel,
        out_type=jax.ShapeDtypeStruct((NP, HIDDEN), jnp.float32),
        mesh=plsc.VectorSubcoreMesh(core_axis_name="c", subcore_axis_name="s"),
        scratch_types=[
            pltpu.VMEM((_DCH,), jnp.int32),
            pltpu.VMEM((_DCH,), jnp.int32),
            pltpu.VMEM((_DCH, HIDDEN), jnp.float32),
            pltpu.VMEM((_DCH, HIDDEN), jnp.float32),
            pltpu.SemaphoreType.DMA,
            pltpu.SemaphoreType.DMA,
            pltpu.SemaphoreType.DMA,
            pltpu.SemaphoreType.DMA,
        ],
    )
    def _sc_dispatch(x_hbm, ppos_hbm, out_hbm, i0, i1, b0, b1, sg0, sg1,
                     sw0, sw1):
        bpw = TOKENS // NW
        wid = lax.axis_index("s") * 2 + lax.axis_index("c")
        base = wid * bpw
        g0 = pltpu.async_copy(x_hbm.at[pl.ds(base, _DCH)], b0, sg0)
        g1 = pltpu.async_copy(x_hbm.at[pl.ds(base + _DCH, _DCH)], b1, sg1)
        pltpu.sync_copy(ppos_hbm.at[pl.ds(base, _DCH)], i0)
        pltpu.sync_copy(ppos_hbm.at[pl.ds(base + _DCH, _DCH)], i1)
        g0.wait()
        w0 = pltpu.async_copy(b0, out_hbm.at[i0], sw0)
        g1.wait()
        w1 = pltpu.async_copy(b1, out_hbm.at[i1], sw1)
        w0.wait()
        w1.wait()

    return _sc_dispatch


_CCH = 16
_CNCH = (TOKENS // NW) // _CCH


@functools.cache
def _make_sc_combine():
    @functools.partial(
        pl.kernel,
        out_type=jax.ShapeDtypeStruct((TOKENS, INTER), jnp.float32),
        mesh=plsc.VectorSubcoreMesh(core_axis_name="c", subcore_axis_name="s"),
        scratch_types=[
            pltpu.VMEM((TOKENS // NW,), jnp.int32),
            pltpu.VMEM((_CCH, INTER), jnp.float32),
            pltpu.VMEM((_CCH, INTER), jnp.float32),
            pltpu.SemaphoreType.DMA,
            pltpu.SemaphoreType.DMA,
            pltpu.SemaphoreType.DMA,
            pltpu.SemaphoreType.DMA,
        ],
    )
    def _sc_combine(src_hbm, g2_hbm, out_hbm, idx_v, b0, b1, sg0, sg1,
                    sw0, sw1):
        bpw = TOKENS // NW
        wid = lax.axis_index("s") * 2 + lax.axis_index("c")
        base = wid * bpw
        pltpu.sync_copy(g2_hbm.at[pl.ds(base, bpw)], idx_v)
        bufs = (b0, b1)
        sg = (sg0, sg1)
        sw = (sw0, sw1)
        gops = [None, None]
        wops = [None, None]
        for c in range(_CNCH):
            b = c & 1
            if wops[b] is not None:
                wops[b].wait()
            gops[b] = pltpu.async_copy(
                src_hbm.at[idx_v.at[pl.ds(c * _CCH, _CCH)]], bufs[b], sg[b])
            if c >= 1:
                pb = (c - 1) & 1
                gops[pb].wait()
                wops[pb] = pltpu.async_copy(
                    bufs[pb], out_hbm.at[pl.ds(base + (c - 1) * _CCH, _CCH)],
                    sw[pb])
        lb = (_CNCH - 1) & 1
        gops[lb].wait()
        wops[lb] = pltpu.async_copy(
            bufs[lb], out_hbm.at[pl.ds(base + (_CNCH - 1) * _CCH, _CCH)],
            sw[lb])
        wops[(_CNCH - 2) & 1].wait()
        wops[lb].wait()

    return _sc_combine


def _dispatch_scatter(x, ppos):
    return _make_sc_dispatch()(x, ppos)


def _combine_gather(src, g2):
    return _make_sc_combine()(src, g2)


def _moe_body(beid_ref, rstart_ref, xs_ref, we_ref, out_ref, wbuf, s0, s1,
              s2, s3):
    w = pl.program_id(0)
    sems = (s0, s1, s2, s3)

    def dma(r):
        return pltpu.make_async_copy(we_ref.at[r], wbuf.at[r % 4],
                                     sems[r % 4])

    @pl.when(w == 0)
    def _():
        dma(0).start()
        dma(1).start()
        dma(2).start()

    for r in range(E):
        @pl.when(w == rstart_ref[r])
        def _(r=r):
            dma(r).wait()
            if r + 3 < E:
                dma(r + 3).start()

    pb = beid_ref[w] % 4
    out_ref[...] = lax.dot_general(
        xs_ref[...], wbuf[pb],
        dimension_numbers=(((1,), (1,)), ((), ())),
        preferred_element_type=jnp.float32,
    )


def _expert_matmul(xs, we, beid, rstart):
    grid_spec = pltpu.PrefetchScalarGridSpec(
        num_scalar_prefetch=2,
        grid=(NBL,),
        in_specs=[
            pl.BlockSpec((B, HIDDEN), lambda w, beid, rstart: (w, 0)),
            pl.BlockSpec(memory_space=pl.ANY),
        ],
        out_specs=pl.BlockSpec((B, INTER), lambda w, beid, rstart: (w, 0)),
        scratch_shapes=[
            pltpu.VMEM((4, INTER, HIDDEN), jnp.float32),
            pltpu.SemaphoreType.DMA,
            pltpu.SemaphoreType.DMA,
            pltpu.SemaphoreType.DMA,
            pltpu.SemaphoreType.DMA,
        ],
    )
    return pl.pallas_call(
        _moe_body,
        grid_spec=grid_spec,
        out_shape=jax.ShapeDtypeStruct((NP, INTER), jnp.float32),
        compiler_params=pltpu.CompilerParams(
            dimension_semantics=("arbitrary",),
        ),
    )(beid, rstart, xs, we)


def kernel(x, Wg, We):
    eidx = _gating(x, Wg)

    oh = (eidx[:, None] == jnp.arange(E, dtype=jnp.int32)[None, :]).astype(
        jnp.int32)
    ccum = jnp.cumsum(oh, axis=0)
    counts = ccum[-1]
    rank = jnp.sum(oh * ccum, axis=1) - 1
    caps = ((counts + B - 1) // B) * B
    pstarts = jnp.concatenate(
        [jnp.zeros((1,), jnp.int32), jnp.cumsum(caps)])[:E]
    ppos = jnp.sum(oh * pstarts[None, :], axis=1) + rank
    beid = (jnp.sum(
        (jnp.arange(NBL, dtype=jnp.int32)[:, None] * B) >= pstarts[None, :],
        axis=1) - 1).astype(jnp.int32)

    rstart = (pstarts // B).astype(jnp.int32)
    xs = _dispatch_scatter(x, ppos)
    out_sorted = _expert_matmul(xs, We, beid, rstart)
    return _combine_gather(out_sorted, ppos)


# speedup vs baseline: 2.0702x; 1.0398x over previous
import functools

import jax
import jax.numpy as jnp
from jax import lax
from jax.experimental import pallas as pl
from jax.experimental.pallas import tpu as pltpu
from jax.experimental.pallas import tpu_sc as plsc

TOKENS = 2048
HIDDEN = 1024
INTER = 2048
E = 8

B = 128
NP = TOKENS + E * B
NBL = NP // B
GBLK = 256
NW = 32


def _gate_body(x_ref, wg_ref, out_ref):
    st = lax.dot_general(
        wg_ref[...], x_ref[...],
        dimension_numbers=(((1,), (1,)), ((), ())),
        preferred_element_type=jnp.float32,
    )
    bv = st[0:1, :]
    bi = jnp.zeros((1, GBLK), jnp.int32)
    for e in range(1, E):
        c = st[e:e + 1, :] > bv
        bi = jnp.where(c, e, bi)
        bv = jnp.where(c, st[e:e + 1, :], bv)
    out_ref[0] = bi


def _gating(x, wg):
    n = TOKENS // GBLK
    out = pl.pallas_call(
        _gate_body,
        grid=(n,),
        in_specs=[
            pl.BlockSpec((GBLK, HIDDEN), lambda w: (w, 0)),
            pl.BlockSpec((E, HIDDEN), lambda w: (0, 0)),
        ],
        out_specs=pl.BlockSpec((1, 1, GBLK), lambda w: (w, 0, 0)),
        out_shape=jax.ShapeDtypeStruct((n, 1, GBLK), jnp.int32),
    )(x, wg)
    return out.reshape(TOKENS)


_DCH = 32


@functools.cache
def _make_sc_dispatch():
    @functools.partial(
        pl.kernel,
        out_type=jax.ShapeDtypeStruct((NP, HIDDEN), jnp.float32),
        mesh=plsc.VectorSubcoreMesh(core_axis_name="c", subcore_axis_name="s"),
        scratch_types=[
            pltpu.VMEM((_DCH,), jnp.int32),
            pltpu.VMEM((_DCH,), jnp.int32),
            pltpu.VMEM((_DCH, HIDDEN), jnp.float32),
            pltpu.VMEM((_DCH, HIDDEN), jnp.float32),
            pltpu.SemaphoreType.DMA,
            pltpu.SemaphoreType.DMA,
            pltpu.SemaphoreType.DMA,
            pltpu.SemaphoreType.DMA,
        ],
    )
    def _sc_dispatch(x_hbm, ppos_hbm, out_hbm, i0, i1, b0, b1, sg0, sg1,
                     sw0, sw1):
        bpw = TOKENS // NW
        wid = lax.axis_index("s") * 2 + lax.axis_index("c")
        base = wid * bpw
        g0 = pltpu.async_copy(x_hbm.at[pl.ds(base, _DCH)], b0, sg0)
        g1 = pltpu.async_copy(x_hbm.at[pl.ds(base + _DCH, _DCH)], b1, sg1)
        pltpu.sync_copy(ppos_hbm.at[pl.ds(base, _DCH)], i0)
        pltpu.sync_copy(ppos_hbm.at[pl.ds(base + _DCH, _DCH)], i1)
        g0.wait()
        w0 = pltpu.async_copy(b0, out_hbm.at[i0], sw0)
        g1.wait()
        w1 = pltpu.async_copy(b1, out_hbm.at[i1], sw1)
        w0.wait()
        w1.wait()

    return _sc_dispatch


_CCH = 16
_CNCH = (TOKENS // NW) // _CCH


@functools.cache
def _make_sc_combine():
    @functools.partial(
        pl.kernel,
        out_type=jax.ShapeDtypeStruct((TOKENS, INTER), jnp.float32),
        mesh=plsc.VectorSubcoreMesh(core_axis_name="c", subcore_axis_name="s"),
        scratch_types=[
            pltpu.VMEM((TOKENS // NW,), jnp.int32),
            pltpu.VMEM((_CCH, INTER), jnp.float32),
            pltpu.VMEM((_CCH, INTER), jnp.float32),
            pltpu.SemaphoreType.DMA,
            pltpu.SemaphoreType.DMA,
            pltpu.SemaphoreType.DMA,
            pltpu.SemaphoreType.DMA,
        ],
    )
    def _sc_combine(src_hbm, g2_hbm, out_hbm, idx_v, b0, b1, sg0, sg1,
                    sw0, sw1):
        bpw = TOKENS // NW
        wid = lax.axis_index("s") * 2 + lax.axis_index("c")
        base = wid * bpw
        pltpu.sync_copy(g2_hbm.at[pl.ds(base, bpw)], idx_v)
        bufs = (b0, b1)
        sg = (sg0, sg1)
        sw = (sw0, sw1)
        gops = [None, None]
        wops = [None, None]
        for c in range(_CNCH):
            b = c & 1
            if wops[b] is not None:
                wops[b].wait()
            gops[b] = pltpu.async_copy(
                src_hbm.at[idx_v.at[pl.ds(c * _CCH, _CCH)]], bufs[b], sg[b])
            if c >= 1:
                pb = (c - 1) & 1
                gops[pb].wait()
                wops[pb] = pltpu.async_copy(
                    bufs[pb], out_hbm.at[pl.ds(base + (c - 1) * _CCH, _CCH)],
                    sw[pb])
        lb = (_CNCH - 1) & 1
        gops[lb].wait()
        wops[lb] = pltpu.async_copy(
            bufs[lb], out_hbm.at[pl.ds(base + (_CNCH - 1) * _CCH, _CCH)],
            sw[lb])
        wops[(_CNCH - 2) & 1].wait()
        wops[lb].wait()

    return _sc_combine


def _dispatch_scatter(x, ppos):
    return _make_sc_dispatch()(x, ppos)


def _combine_gather(src, g2):
    return _make_sc_combine()(src, g2)


def _moe_body(beid_ref, rstart_ref, xs_ref, we_ref, out_ref, wbuf, s0, s1,
              s2, s3):
    w = pl.program_id(0)
    sems = (s0, s1, s2, s3)

    def dma(r):
        return pltpu.make_async_copy(we_ref.at[r], wbuf.at[r % 4],
                                     sems[r % 4])

    @pl.when(w == 0)
    def _():
        dma(0).start()
        dma(1).start()
        dma(2).start()

    for r in range(E):
        @pl.when(w == rstart_ref[r])
        def _(r=r):
            dma(r).wait()
            if r + 3 < E:
                dma(r + 3).start()

    pb = beid_ref[w] % 4

    @pl.when(w < rstart_ref[E])
    def _():
        out_ref[...] = lax.dot_general(
            xs_ref[...], wbuf[pb],
            dimension_numbers=(((1,), (1,)), ((), ())),
            preferred_element_type=jnp.float32,
        )


def _expert_matmul(xs, we, beid, rstart):
    grid_spec = pltpu.PrefetchScalarGridSpec(
        num_scalar_prefetch=2,
        grid=(NBL,),
        in_specs=[
            pl.BlockSpec((B, HIDDEN), lambda w, beid, rstart: (w, 0)),
            pl.BlockSpec(memory_space=pl.ANY),
        ],
        out_specs=pl.BlockSpec((B, INTER), lambda w, beid, rstart: (w, 0)),
        scratch_shapes=[
            pltpu.VMEM((4, INTER, HIDDEN), jnp.float32),
            pltpu.SemaphoreType.DMA,
            pltpu.SemaphoreType.DMA,
            pltpu.SemaphoreType.DMA,
            pltpu.SemaphoreType.DMA,
        ],
    )
    return pl.pallas_call(
        _moe_body,
        grid_spec=grid_spec,
        out_shape=jax.ShapeDtypeStruct((NP, INTER), jnp.float32),
        compiler_params=pltpu.CompilerParams(
            dimension_semantics=("arbitrary",),
        ),
    )(beid, rstart, xs, we)


def kernel(x, Wg, We):
    eidx = _gating(x, Wg)

    oh = (eidx[:, None] == jnp.arange(E, dtype=jnp.int32)[None, :]).astype(
        jnp.int32)
    ccum = jnp.cumsum(oh, axis=0)
    counts = ccum[-1]
    rank = jnp.sum(oh * ccum, axis=1) - 1
    caps = ((counts + B - 1) // B) * B
    pstarts = jnp.concatenate(
        [jnp.zeros((1,), jnp.int32), jnp.cumsum(caps)])[:E]
    ppos = jnp.sum(oh * pstarts[None, :], axis=1) + rank
    beid = (jnp.sum(
        (jnp.arange(NBL, dtype=jnp.int32)[:, None] * B) >= pstarts[None, :],
        axis=1) - 1).astype(jnp.int32)

    rstart = (jnp.concatenate(
        [pstarts, (pstarts[7:] + caps[7:])]) // B).astype(jnp.int32)
    xs = _dispatch_scatter(x, ppos)
    out_sorted = _expert_matmul(xs, We, beid, rstart)
    return _combine_gather(out_sorted, ppos)
